# baseline (speedup 1.0000x reference)
import functools

import jax
import jax.numpy as jnp
from jax import lax
from jax.experimental import pallas as pl
from jax.experimental.pallas import tpu as pltpu
from jax.experimental.pallas import tpu_sc as plsc

_VOCAB = 100000
_DIM = 64
_SEQ = 50
_BATCH = 4096
_LANES = 16

_PROWS = 56
_VPAD = 100352
_BN = 3584
_TSTACK = 4

_NC = 2
_NS = 16
_NW = _NC * _NS

_CR = 16
_IDX_PER_CHUNK = _CR * _SEQ
_GPIECE = 80
_NPIECE = _IDX_PER_CHUNK // _GPIECE
_NCHUNKS = _BATCH // _CR
_CPW = _NCHUNKS // _NW


_NT = _BN // 128
_VT = _VPAD // 128


def _pmat_body(w4_ref, et_ref, o_ref):
  w4 = w4_ref[...]
  for g in range(_NT // _TSTACK):
    e4 = jnp.concatenate(
        [et_ref[:, (_TSTACK * g + k) * 128:(_TSTACK * g + k + 1) * 128]
         for k in range(_TSTACK)], axis=0)
    m = jnp.dot(w4, e4, preferred_element_type=jnp.float32)
    o_ref[pl.ds(_TSTACK * g, _TSTACK)] = m.reshape(_TSTACK, _PROWS, 128)


def _tc_pmat(w4, emb_t):
  grid = _VT // _NT
  return pl.pallas_call(
      _pmat_body,
      grid=(grid,),
      in_specs=[
          pl.BlockSpec((_TSTACK * _PROWS, _TSTACK * _DIM), lambda i: (0, 0)),
          pl.BlockSpec((_DIM, _BN), lambda i: (0, i)),
      ],
      out_specs=pl.BlockSpec((_NT, _PROWS, 128), lambda i: (i, 0, 0)),
      out_shape=jax.ShapeDtypeStruct((_VT, _PROWS, 128), jnp.float32),
  )(w4, emb_t)


def _sc_gather_sum(xq, p1d, bvec):
  mesh = plsc.VectorSubcoreMesh(core_axis_name="c", subcore_axis_name="s")

  @functools.partial(
      pl.kernel,
      out_type=jax.ShapeDtypeStruct((_BATCH,), jnp.float32),
      mesh=mesh,
      compiler_params=pltpu.CompilerParams(
          needs_layout_passes=False, use_tc_tiling_on_sc=False),
      scratch_types=[
          pltpu.VMEM((_NPIECE, _GPIECE), jnp.int32),
          pltpu.VMEM((_NPIECE, _GPIECE), jnp.int32),
          pltpu.VMEM((_IDX_PER_CHUNK,), jnp.float32),
          pltpu.VMEM((_IDX_PER_CHUNK,), jnp.float32),
          pltpu.VMEM((_CPW * _CR,), jnp.float32),
          pltpu.VMEM((_LANES,), jnp.float32),
          pltpu.SemaphoreType.DMA,
          pltpu.SemaphoreType.DMA,
      ],
  )
  def k(xq_hbm, p_hbm, bvec_hbm, out_hbm,
        idx0, idx1, g0, g1, outst_v, bvec_v, s0, s1):
    wid = lax.axis_index("s") * _NC + lax.axis_index("c")
    first = wid * _CPW
    idx_bufs = (idx0, idx1)
    g_bufs = (g0, g1)
    sems = (s0, s1)

    pltpu.sync_copy(bvec_hbm, bvec_v)
    bv = bvec_v[...]

    def issue(ck, buf):
      pltpu.sync_copy(xq_hbm.at[ck], idx_bufs[buf])
      for j in range(_NPIECE):
        pltpu.async_copy(
            p_hbm.at[idx_bufs[buf].at[j]],
            g_bufs[buf].at[pl.ds(j * _GPIECE, _GPIECE)],
            sems[buf])

    def drain(buf):
      pltpu.make_async_copy(
          p_hbm.at[pl.ds(0, _IDX_PER_CHUNK)], g_bufs[buf], sems[buf]).wait()

    def compute(g, buf):
      gv = g_bufs[buf]
      ov = bv
      for l in range(_SEQ):
        ov = ov + gv[pl.ds(l * _LANES, _LANES)]
      outst_v[pl.ds(g * _CR, _CR)] = ov

    issue(first, 0)

    def step(t, _):
      gch = t * 2
      issue(first + gch + 1, 1)
      drain(0)
      compute(gch, 0)

      @pl.when(gch + 2 < _CPW)
      def _():
        issue(first + gch + 2, 0)

      drain(1)
      compute(gch + 1, 1)
      return 0

    lax.fori_loop(0, _CPW // 2, step, 0)

    pltpu.sync_copy(outst_v, out_hbm.at[pl.ds(wid * (_CPW * _CR), _CPW * _CR)])

  return k(xq, p1d, bvec)


def kernel(x, emb, W, b):
  wr = W.reshape(_SEQ, _DIM)
  w4 = jnp.zeros((_TSTACK * _PROWS, _TSTACK * _DIM), jnp.float32)
  for tt in range(_TSTACK):
    w4 = w4.at[tt * _PROWS:tt * _PROWS + _SEQ,
               tt * _DIM:(tt + 1) * _DIM].set(wr)
  pt = _tc_pmat(w4, emb.T)
  p1d = pt.reshape(-1)

  xi = x.astype(jnp.int32)
  xe = ((xi >> 7) * (_PROWS * 128) + (xi & 127)
        + 128 * jnp.arange(_SEQ, dtype=jnp.int32)[None, :])
  xq = xe.reshape(_NCHUNKS, _CR, _SEQ).transpose(0, 2, 1).reshape(
      _NCHUNKS, _NPIECE, _GPIECE)

  bvec = jnp.broadcast_to(b.astype(jnp.float32), (_LANES,))
  return _sc_gather_sum(xq, p1d, bvec)


# speedup vs baseline: 1.0995x; 1.0995x over previous
import functools

import jax
import jax.numpy as jnp
from jax import lax
from jax.experimental import pallas as pl
from jax.experimental.pallas import tpu as pltpu
from jax.experimental.pallas import tpu_sc as plsc

_VOCAB = 100000
_DIM = 64
_SEQ = 50
_BATCH = 4096
_LANES = 16

_PROWS = 56
_VPAD = 100096
_BN = 4352

_NC = 2
_NS = 16
_NW = _NC * _NS

_CR = 16
_IDX_PER_CHUNK = _CR * _SEQ
_GPIECE = 80
_NPIECE = _IDX_PER_CHUNK // _GPIECE
_NCHUNKS = _BATCH // _CR
_CPW = _NCHUNKS // _NW


_NT = _BN // 128
_VT = _VPAD // 128


def _pmat_body(w_ref, et_ref, o_ref):
  w = w_ref[...]
  for t in range(_NT):
    o_ref[t] = jnp.dot(w, et_ref[:, t * 128:(t + 1) * 128],
                       preferred_element_type=jnp.float32)


def _tc_pmat(w2t, emb_t):
  grid = _VT // _NT
  return pl.pallas_call(
      _pmat_body,
      grid=(grid,),
      in_specs=[
          pl.BlockSpec((_PROWS, _DIM), lambda i: (0, 0)),
          pl.BlockSpec((_DIM, _BN), lambda i: (0, i)),
      ],
      out_specs=pl.BlockSpec((_NT, _PROWS, 128), lambda i: (i, 0, 0)),
      out_shape=jax.ShapeDtypeStruct((_VT, _PROWS, 128), jnp.float32),
  )(w2t, emb_t)


def _sc_gather_sum(xq, p1d, bvec):
  mesh = plsc.VectorSubcoreMesh(core_axis_name="c", subcore_axis_name="s")

  @functools.partial(
      pl.kernel,
      out_type=jax.ShapeDtypeStruct((_BATCH,), jnp.float32),
      mesh=mesh,
      compiler_params=pltpu.CompilerParams(
          needs_layout_passes=False, use_tc_tiling_on_sc=False),
      scratch_types=[
          pltpu.VMEM((_NPIECE, _GPIECE), jnp.int32),
          pltpu.VMEM((_NPIECE, _GPIECE), jnp.int32),
          pltpu.VMEM((_IDX_PER_CHUNK,), jnp.float32),
          pltpu.VMEM((_IDX_PER_CHUNK,), jnp.float32),
          pltpu.VMEM((_CPW * _CR,), jnp.float32),
          pltpu.VMEM((_LANES,), jnp.float32),
          pltpu.SemaphoreType.DMA,
          pltpu.SemaphoreType.DMA,
      ],
  )
  def k(xq_hbm, p_hbm, bvec_hbm, out_hbm,
        idx0, idx1, g0, g1, outst_v, bvec_v, s0, s1):
    wid = lax.axis_index("s") * _NC + lax.axis_index("c")
    first = wid * _CPW
    idx_bufs = (idx0, idx1)
    g_bufs = (g0, g1)
    sems = (s0, s1)

    pltpu.sync_copy(bvec_hbm, bvec_v)
    bv = bvec_v[...]

    def issue(ck, buf):
      pltpu.sync_copy(xq_hbm.at[ck], idx_bufs[buf])
      for j in range(_NPIECE):
        pltpu.async_copy(
            p_hbm.at[idx_bufs[buf].at[j]],
            g_bufs[buf].at[pl.ds(j * _GPIECE, _GPIECE)],
            sems[buf])

    def drain(buf):
      pltpu.make_async_copy(
          p_hbm.at[pl.ds(0, _IDX_PER_CHUNK)], g_bufs[buf], sems[buf]).wait()

    def compute(g, buf):
      gv = g_bufs[buf]
      ov = bv
      for l in range(_SEQ):
        ov = ov + gv[pl.ds(l * _LANES, _LANES)]
      outst_v[pl.ds(g * _CR, _CR)] = ov

    issue(first, 0)

    def step(t, _):
      gch = t * 2
      issue(first + gch + 1, 1)
      drain(0)
      compute(gch, 0)

      @pl.when(gch + 2 < _CPW)
      def _():
        issue(first + gch + 2, 0)

      drain(1)
      compute(gch + 1, 1)
      return 0

    lax.fori_loop(0, _CPW // 2, step, 0)

    pltpu.sync_copy(outst_v, out_hbm.at[pl.ds(wid * (_CPW * _CR), _CPW * _CR)])

  return k(xq, p1d, bvec)


def kernel(x, emb, W, b):
  wr = W.reshape(_SEQ, _DIM)
  w2t = jnp.zeros((_PROWS, _DIM), jnp.float32).at[:_SEQ].set(wr)
  pt = _tc_pmat(w2t, emb.T)
  p1d = pt.reshape(-1)

  xi = x.astype(jnp.int32)
  xe = ((xi >> 7) * (_PROWS * 128) + (xi & 127)
        + 128 * jnp.arange(_SEQ, dtype=jnp.int32)[None, :])
  xq = xe.reshape(_NCHUNKS, _CR, _SEQ).transpose(0, 2, 1).reshape(
      _NCHUNKS, _NPIECE, _GPIECE)

  bvec = jnp.broadcast_to(b.astype(jnp.float32), (_LANES,))
  return _sc_gather_sum(xq, p1d, bvec)


# speedup vs baseline: 1.1227x; 1.0211x over previous
import functools

import jax
import jax.numpy as jnp
from jax import lax
from jax.experimental import pallas as pl
from jax.experimental.pallas import tpu as pltpu
from jax.experimental.pallas import tpu_sc as plsc

_VOCAB = 100000
_DIM = 64
_SEQ = 50
_BATCH = 4096
_LANES = 16

_PROWS = 56
_VPAD = 100096
_BN = 4352

_NC = 2
_NS = 16
_NW = _NC * _NS

_CR = 16
_IDX_PER_CHUNK = _CR * _SEQ
_GPIECE = 80
_NPIECE = _IDX_PER_CHUNK // _GPIECE
_NCHUNKS = _BATCH // _CR
_CPW = _NCHUNKS // _NW


_NT = _BN // 128
_VT = _VPAD // 128


def _pmat_body(w_ref, et_ref, o_ref):
  m = jnp.dot(w_ref[...], et_ref[...], preferred_element_type=jnp.float32)
  for t in range(_NT):
    o_ref[t] = m[:, t * 128:(t + 1) * 128]


def _tc_pmat(w2t, emb_t):
  grid = _VT // _NT
  return pl.pallas_call(
      _pmat_body,
      grid=(grid,),
      in_specs=[
          pl.BlockSpec((_PROWS, _DIM), lambda i: (0, 0)),
          pl.BlockSpec((_DIM, _BN), lambda i: (0, i)),
      ],
      out_specs=pl.BlockSpec((_NT, _PROWS, 128), lambda i: (i, 0, 0)),
      out_shape=jax.ShapeDtypeStruct((_VT, _PROWS, 128), jnp.float32),
  )(w2t, emb_t)


def _sc_gather_sum(xq, p1d, bvec):
  mesh = plsc.VectorSubcoreMesh(core_axis_name="c", subcore_axis_name="s")

  @functools.partial(
      pl.kernel,
      out_type=jax.ShapeDtypeStruct((_BATCH,), jnp.float32),
      mesh=mesh,
      compiler_params=pltpu.CompilerParams(
          needs_layout_passes=False, use_tc_tiling_on_sc=False),
      scratch_types=[
          pltpu.VMEM((_NPIECE, _GPIECE), jnp.int32),
          pltpu.VMEM((_NPIECE, _GPIECE), jnp.int32),
          pltpu.VMEM((_IDX_PER_CHUNK,), jnp.float32),
          pltpu.VMEM((_IDX_PER_CHUNK,), jnp.float32),
          pltpu.VMEM((_CPW * _CR,), jnp.float32),
          pltpu.VMEM((_LANES,), jnp.float32),
          pltpu.SemaphoreType.DMA,
          pltpu.SemaphoreType.DMA,
      ],
  )
  def k(xq_hbm, p_hbm, bvec_hbm, out_hbm,
        idx0, idx1, g0, g1, outst_v, bvec_v, s0, s1):
    wid = lax.axis_index("s") * _NC + lax.axis_index("c")
    first = wid * _CPW
    idx_bufs = (idx0, idx1)
    g_bufs = (g0, g1)
    sems = (s0, s1)

    pltpu.sync_copy(bvec_hbm, bvec_v)
    bv = bvec_v[...]

    def issue(ck, buf):
      pltpu.sync_copy(xq_hbm.at[ck], idx_bufs[buf])
      for j in range(_NPIECE):
        pltpu.async_copy(
            p_hbm.at[idx_bufs[buf].at[j]],
            g_bufs[buf].at[pl.ds(j * _GPIECE, _GPIECE)],
            sems[buf])

    def drain(buf):
      pltpu.make_async_copy(
          p_hbm.at[pl.ds(0, _IDX_PER_CHUNK)], g_bufs[buf], sems[buf]).wait()

    def compute(g, buf):
      gv = g_bufs[buf]
      ov = bv
      for l in range(_SEQ):
        ov = ov + gv[pl.ds(l * _LANES, _LANES)]
      outst_v[pl.ds(g * _CR, _CR)] = ov

    issue(first, 0)

    def step(t, _):
      gch = t * 2
      issue(first + gch + 1, 1)
      drain(0)
      compute(gch, 0)

      @pl.when(gch + 2 < _CPW)
      def _():
        issue(first + gch + 2, 0)

      drain(1)
      compute(gch + 1, 1)
      return 0

    lax.fori_loop(0, _CPW // 2, step, 0)

    pltpu.sync_copy(outst_v, out_hbm.at[pl.ds(wid * (_CPW * _CR), _CPW * _CR)])

  return k(xq, p1d, bvec)


def kernel(x, emb, W, b):
  wr = W.reshape(_SEQ, _DIM)
  w2t = jnp.zeros((_PROWS, _DIM), jnp.float32).at[:_SEQ].set(wr)
  pt = _tc_pmat(w2t, emb.T)
  p1d = pt.reshape(-1)

  xi = x.astype(jnp.int32)
  xe = ((xi >> 7) * (_PROWS * 128) + (xi & 127)
        + 128 * jnp.arange(_SEQ, dtype=jnp.int32)[None, :])
  xq = xe.reshape(_NCHUNKS, _CR, _SEQ).transpose(0, 2, 1).reshape(
      _NCHUNKS, _NPIECE, _GPIECE)

  bvec = jnp.broadcast_to(b.astype(jnp.float32), (_LANES,))
  return _sc_gather_sum(xq, p1d, bvec)


# speedup vs baseline: 1.2068x; 1.0749x over previous
import functools

import jax
import jax.numpy as jnp
from jax import lax
from jax.experimental import pallas as pl
from jax.experimental.pallas import tpu as pltpu
from jax.experimental.pallas import tpu_sc as plsc

_VOCAB = 100000
_DIM = 64
_SEQ = 50
_BATCH = 4096
_LANES = 16

_PROWS = 56
_VPAD = 100096
_BN = 5888

_NC = 2
_NS = 16
_NW = _NC * _NS

_CR = 16
_IDX_PER_CHUNK = _CR * _SEQ
_GPIECE = 80
_NPIECE = _IDX_PER_CHUNK // _GPIECE
_NCHUNKS = _BATCH // _CR
_CPW = _NCHUNKS // _NW


_NT = _BN // 128
_VT = _VPAD // 128


def _pmat_body(w_ref, et_ref, o_ref):
  m = jnp.dot(w_ref[...], et_ref[...], preferred_element_type=jnp.float32)
  for t in range(_NT):
    o_ref[t] = m[:, t * 128:(t + 1) * 128]


def _tc_pmat(w2t, emb_t):
  grid = _VT // _NT
  return pl.pallas_call(
      _pmat_body,
      grid=(grid,),
      in_specs=[
          pl.BlockSpec((_PROWS, _DIM), lambda i: (0, 0)),
          pl.BlockSpec((_DIM, _BN), lambda i: (0, i)),
      ],
      out_specs=pl.BlockSpec((_NT, _PROWS, 128), lambda i: (i, 0, 0)),
      out_shape=jax.ShapeDtypeStruct((_VT, _PROWS, 128), jnp.float32),
  )(w2t, emb_t)


def _sc_gather_sum(xq, p1d, bvec):
  mesh = plsc.VectorSubcoreMesh(core_axis_name="c", subcore_axis_name="s")

  @functools.partial(
      pl.kernel,
      out_type=jax.ShapeDtypeStruct((_BATCH,), jnp.float32),
      mesh=mesh,
      compiler_params=pltpu.CompilerParams(
          needs_layout_passes=False, use_tc_tiling_on_sc=False),
      scratch_types=[
          pltpu.VMEM((_NPIECE, _GPIECE), jnp.int32),
          pltpu.VMEM((_NPIECE, _GPIECE), jnp.int32),
          pltpu.VMEM((_IDX_PER_CHUNK,), jnp.float32),
          pltpu.VMEM((_IDX_PER_CHUNK,), jnp.float32),
          pltpu.VMEM((_CPW * _CR,), jnp.float32),
          pltpu.VMEM((_LANES,), jnp.float32),
          pltpu.SemaphoreType.DMA,
          pltpu.SemaphoreType.DMA,
      ],
  )
  def k(xq_hbm, p_hbm, bvec_hbm, out_hbm,
        idx0, idx1, g0, g1, outst_v, bvec_v, s0, s1):
    wid = lax.axis_index("s") * _NC + lax.axis_index("c")
    first = wid * _CPW
    idx_bufs = (idx0, idx1)
    g_bufs = (g0, g1)
    sems = (s0, s1)

    pltpu.sync_copy(bvec_hbm, bvec_v)
    bv = bvec_v[...]

    def issue(ck, buf):
      pltpu.sync_copy(xq_hbm.at[ck], idx_bufs[buf])
      for j in range(_NPIECE):
        pltpu.async_copy(
            p_hbm.at[idx_bufs[buf].at[j]],
            g_bufs[buf].at[pl.ds(j * _GPIECE, _GPIECE)],
            sems[buf])

    def drain(buf):
      pltpu.make_async_copy(
          p_hbm.at[pl.ds(0, _IDX_PER_CHUNK)], g_bufs[buf], sems[buf]).wait()

    def compute(g, buf):
      gv = g_bufs[buf]
      ov = bv
      for l in range(_SEQ):
        ov = ov + gv[pl.ds(l * _LANES, _LANES)]
      outst_v[pl.ds(g * _CR, _CR)] = ov

    issue(first, 0)

    def step(t, _):
      gch = t * 2
      issue(first + gch + 1, 1)
      drain(0)
      compute(gch, 0)

      @pl.when(gch + 2 < _CPW)
      def _():
        issue(first + gch + 2, 0)

      drain(1)
      compute(gch + 1, 1)
      return 0

    lax.fori_loop(0, _CPW // 2, step, 0)

    pltpu.sync_copy(outst_v, out_hbm.at[pl.ds(wid * (_CPW * _CR), _CPW * _CR)])

  return k(xq, p1d, bvec)


def kernel(x, emb, W, b):
  wr = W.reshape(_SEQ, _DIM)
  w2t = jnp.zeros((_PROWS, _DIM), jnp.float32).at[:_SEQ].set(wr)
  pt = _tc_pmat(w2t, emb.T)
  p1d = pt.reshape(-1)

  xi = x.astype(jnp.int32)
  xe = ((xi >> 7) * (_PROWS * 128) + (xi & 127)
        + 128 * jnp.arange(_SEQ, dtype=jnp.int32)[None, :])
  xq = xe.reshape(_NCHUNKS, _CR, _SEQ).transpose(0, 2, 1).reshape(
      _NCHUNKS, _NPIECE, _GPIECE)

  bvec = jnp.broadcast_to(b.astype(jnp.float32), (_LANES,))
  return _sc_gather_sum(xq, p1d, bvec)
